# Initial kernel scaffold; baseline (speedup 1.0000x reference)
import jax
import jax.numpy as jnp
from jax.experimental import pallas as pl


def kernel(x, edge_index, batch, W0, b0, W1, b1, W2, b2, Wlin, blin):
    raise NotImplementedError("write your pallas kernel here")


# speedup vs baseline: 8.8294x; 8.8294x over previous
import functools

import jax
import jax.numpy as jnp
from jax import lax
from jax.experimental import pallas as pl
from jax.experimental.pallas import tpu as pltpu
from jax.experimental.pallas import tpu_sc as plsc

N = 10000
E = 320000
F_IN = 128
H = 128
C = 16
G = 64

NC = 2
NS = 16
NW = NC * NS
CH = 128
K = -(-E // (NW * CH))
EPT = K * CH
E_PAD = NW * EPT
ACC_ROWS = 10240
ZCH = ACC_ROWS // NS // CH
DW = 16

_mesh = plsc.VectorSubcoreMesh(
    core_axis_name="c", subcore_axis_name="s", num_cores=NC, num_subcores=NS)

_f32 = jnp.float32


def _agg_body(xs_hbm, srcg_hbm, dstg_hbm, zeros_hbm, out_hbm,
              src_v, dst_v, rows_v, acc_sh, sem):
  cid = lax.axis_index("c")
  sid = lax.axis_index("s")
  wid = sid * NC + cid
  pltpu.sync_copy(zeros_hbm, rows_v)
  pltpu.sync_copy(srcg_hbm.at[wid], src_v)
  pltpu.sync_copy(dstg_hbm.at[wid], dst_v)

  def zacc(t, _):
    pltpu.sync_copy(rows_v, acc_sh.at[pl.ds((sid * ZCH + t) * CH, CH)])
    return 0
  lax.fori_loop(0, ZCH, zacc, 0)
  plsc.subcore_barrier()

  def step(j, _):
    pltpu.async_copy(xs_hbm.at[src_v.at[j]], rows_v, sem).wait()
    pltpu.sync_copy(rows_v, acc_sh.at[dst_v.at[j]], add=True)
    return 0
  lax.fori_loop(0, K, step, 0)
  plsc.subcore_barrier()

  def cout(t, _):
    r0 = (sid * ZCH + t) * CH
    pltpu.sync_copy(acc_sh.at[pl.ds(r0, CH)], rows_v)
    pltpu.sync_copy(rows_v, out_hbm.at[cid, pl.ds(r0, CH)])
    return 0
  lax.fori_loop(0, ZCH, cout, 0)


_agg_call = pl.kernel(
    _agg_body,
    out_type=jax.ShapeDtypeStruct((NC, ACC_ROWS, H), _f32),
    mesh=_mesh,
    scratch_types=[
        pltpu.VMEM((K, CH), jnp.int32),
        pltpu.VMEM((K, CH), jnp.int32),
        pltpu.VMEM((CH, H), _f32),
        pltpu.VMEM_SHARED((ACC_ROWS, H), _f32),
        pltpu.SemaphoreType.DMA,
    ],
)

_DOT = dict(preferred_element_type=_f32, precision=lax.Precision.HIGHEST)


def _tc0_body(degp_ref, x_ref, w0_ref, dinv_ref, xs0_ref):
  deg = degp_ref[0, :N, 0:1] + degp_ref[1, :N, 0:1] + 1.0
  dinv = lax.rsqrt(deg)
  dinv_ref[...] = dinv
  xw = jnp.dot(x_ref[...], w0_ref[...], **_DOT)
  xs0_ref[...] = xw * dinv


def _tc0_call(degp, x, w0):
  return pl.pallas_call(
      _tc0_body,
      out_shape=[jax.ShapeDtypeStruct((N, 1), _f32),
                 jax.ShapeDtypeStruct((N, H), _f32)],
  )(degp, x, w0)


def _tc_mid_body(relu, sp_ref, xs_ref, dinv_ref, b_ref, w_ref, out_ref):
  dinv = dinv_ref[...]
  h = dinv * (sp_ref[0, :N] + sp_ref[1, :N] + xs_ref[...]) + b_ref[...]
  if relu:
    h = jnp.maximum(h, 0.0)
  out_ref[...] = dinv * jnp.dot(h, w_ref[...], **_DOT)


def _tc_mid_call(relu, sp, xs, dinv, b, w):
  return pl.pallas_call(
      functools.partial(_tc_mid_body, relu),
      out_shape=jax.ShapeDtypeStruct((N, H), _f32),
  )(sp, xs, dinv, b, w)


def _tc3_body(sp_ref, xs_ref, dinv_ref, b_ref, batch_ref, wlin_ref, blin_ref,
              y_ref, gm_ref):
  dinv = dinv_ref[...]
  h = jnp.maximum(dinv * (sp_ref[0, :N] + sp_ref[1, :N] + xs_ref[...]) + b_ref[...],
                  0.0)
  gids = batch_ref[...]
  seg_ids = lax.broadcasted_iota(jnp.int32, (1, G), 1)
  onehot = (gids == seg_ids).astype(_f32)
  seg = lax.dot_general(onehot, h, (((0,), (0,)), ((), ())), **_DOT)
  cnt = lax.dot_general(onehot, jnp.ones((N, 1), _f32),
                        (((0,), (0,)), ((), ())), **_DOT)
  gm = seg / jnp.maximum(cnt, 1.0)
  gm_ref[...] = gm
  y_ref[...] = jnp.dot(gm, wlin_ref[...], **_DOT) + blin_ref[...]


def _tc3_call(sp, xs, dinv, b, batch2d, wlin, blin):
  return pl.pallas_call(
      _tc3_body,
      out_shape=[jax.ShapeDtypeStruct((G, C), _f32),
                 jax.ShapeDtypeStruct((G, H), _f32)],
  )(sp, xs, dinv, b, batch2d, wlin, blin)


def kernel(x, edge_index, batch, W0, b0, W1, b1, W2, b2, Wlin, blin):
  src = edge_index[0]
  dst = edge_index[1]
  pad = E_PAD - E
  srcg = jnp.concatenate([src, jnp.zeros((pad,), jnp.int32)]).reshape(NW, K, CH)
  dstg = jnp.concatenate([dst, jnp.full((pad,), N, jnp.int32)]).reshape(NW, K, CH)
  zrows = jnp.zeros((CH, H), _f32)

  degp = _agg_call(jnp.ones((N, H), _f32), srcg, dstg, zrows)
  dinv, xs = _tc0_call(degp, x, W0)
  for b, w, relu in ((b0, W1, False), (b1, W2, True)):
    sp = _agg_call(xs, srcg, dstg, zrows)
    xs = _tc_mid_call(relu, sp, xs, dinv, b.reshape(1, H), w)
  sp = _agg_call(xs, srcg, dstg, zrows)
  y, gm = _tc3_call(sp, xs, dinv, b2.reshape(1, H), batch.reshape(N, 1),
                    Wlin, blin.reshape(1, C))
  return (y, gm)
